# baseline (speedup 1.0000x reference)
import functools

import jax
import jax.numpy as jnp
import numpy as np
from jax import lax
from jax.experimental import pallas as pl
from jax.experimental.pallas import tpu as pltpu
from jax.experimental.pallas import tpu_sc as plsc

_NPOINTS = 1024
_VS = 16
_V = _VS ** 3
_P = 8
_B = 16
_NW = 32
_CHUNK = _B * _NPOINTS // _NW
_JROWS = _CHUNK // 128


def _threefry2x32(kpair, count):
    odd = count.size % 2
    flat = count.astype(np.uint32).ravel()
    if odd:
        flat = np.concatenate([flat, np.zeros(1, np.uint32)])
    half = flat.size // 2
    x0, x1 = flat[:half].copy(), flat[half:].copy()
    k0, k1 = np.uint32(kpair[0]), np.uint32(kpair[1])
    ks = [k0, k1, k0 ^ k1 ^ np.uint32(0x1BD11BDA)]
    rotations = ((13, 15, 26, 6), (17, 29, 16, 24))
    with np.errstate(over="ignore"):
        x0 += ks[0]
        x1 += ks[1]
        for i in range(5):
            for r in rotations[i % 2]:
                x0 += x1
                x1 = (x1 << np.uint32(r)) | (x1 >> np.uint32(32 - r))
                x1 ^= x0
            x0 += ks[(i + 1) % 3]
            x1 += ks[(i + 2) % 3] + np.uint32(i + 1)
    out = np.concatenate([x0, x1])
    if odd:
        out = out[:-1]
    return out.reshape(count.shape)


def _fry_blocks(kpair, n):
    cnt = np.concatenate([np.zeros(n, np.uint32), np.arange(n, dtype=np.uint32)])
    out = _threefry2x32(kpair, cnt)
    return out[:n], out[n:]


def _random_bits(kpair, shape):
    size = int(np.prod(shape))
    hi, lo = _fry_blocks(kpair, size)
    return (hi ^ lo).reshape(shape)


def _split(kpair, n=2):
    hi, lo = _fry_blocks(kpair, n)
    return np.stack([hi, lo], axis=-1)


@functools.lru_cache(maxsize=1)
def _rvs_constants():
    root = (np.uint32(0), np.uint32(42))
    keys = _split(root, _B)
    sel = np.empty((_B, _NPOINTS), np.int32)
    r = np.empty((_B, _NPOINTS), np.int32)
    span = np.uint32(2 ** 31 - 1)
    with np.errstate(over="ignore"):
        m16 = np.uint32(np.uint32(65536) % span)
        mult = np.uint32(np.uint32(m16 * m16) % span)
    for b in range(_B):
        kv, kp = _split(keys[b])
        bits = _random_bits(kv, (_V,))
        scores = ((bits >> np.uint32(9)) | np.uint32(0x3F800000)).view(
            np.float32) - np.float32(1.0)
        order = np.argsort(scores, kind="stable")
        sel[b] = order[:_NPOINTS].astype(np.int32)
        khi, klo = _split(kp)
        u = _random_bits(khi, (_NPOINTS,))
        v = _random_bits(klo, (_NPOINTS,))
        with np.errstate(over="ignore"):
            off = np.uint32((u % span) * mult + v % span) % span
        r[b] = off.astype(np.int32)

    bb = np.arange(_B, dtype=np.int32)[:, None]
    phys = (((bb // 8) * (_V // 128) + sel // 128) * 1024
            + (bb % 8) * 128 + sel % 128)
    slotpack = np.zeros_like(r)
    for c in range(1, _P + 1):
        slotpack |= (r % c) << (3 * (c - 1))
    const_rows = np.stack([phys.reshape(_NW, _CHUNK),
                           sel.reshape(_NW, _CHUNK),
                           slotpack.reshape(_NW, _CHUNK)], axis=1)

    kx = sel // (_VS * _VS)
    ky = (sel // _VS) % _VS
    kz = sel % _VS
    bids = np.broadcast_to(np.arange(_B, dtype=np.int32)[:, None], (_B, _NPOINTS))
    cidx = np.stack([bids, kx, ky, kz], axis=-1).astype(np.int32)
    const_tab = const_rows.reshape(_NW, 3 * _CHUNK)
    cidx4 = cidx.reshape(_B, _NPOINTS // 128, 128, 4).transpose(0, 1, 3, 2)
    cidx4 = np.ascontiguousarray(cidx4)
    return const_tab, cidx, cidx4


def _rvs_body(vc_hbm, vp_hbm, const_hbm, cidx_hbm, out_hbm, oidx_hbm,
              cr_v, vcs_v, fidx_v, cent_v, sem_k, sem_v, sem_o, sem_i, *sems):
    wid = lax.axis_index("s") * 2 + lax.axis_index("c")
    b = wid // 2
    h = wid - 2 * b

    ci = pltpu.async_copy(cidx_hbm.at[b, pl.ds(h * _JROWS, _JROWS)],
                          oidx_hbm.at[b, pl.ds(h * _JROWS, _JROWS)], sem_i)

    ck = pltpu.async_copy(const_hbm.at[wid], cr_v, sem_k)
    cv = pltpu.async_copy(vc_hbm.at[b], vcs_v, sem_v)
    ck.wait()
    cv.wait()

    cent_copies = []
    for j in range(_JROWS):
        for i in range(8):
            pt = j * 128 + i * 16
            cnt = plsc.load_gather(vcs_v, [cr_v[pl.ds(_CHUNK + pt, 16)]])
            pack = cr_v[pl.ds(2 * _CHUNK + pt, 16)]
            slot = lax.shift_right_logical(pack, cnt * 3 - 3) & 7
            fidx_v[j, pl.ds(i * 16, 16)] = (slot * (_B * _V)
                                            + cr_v[pl.ds(pt, 16)])
        cent_copies.append(
            pltpu.async_copy(vp_hbm.at[fidx_v.at[j]], cent_v.at[j], sems[j]))

    out_rows = []
    for j in range(_JROWS):
        cent_copies[j].wait()
        out_rows.append(
            pltpu.async_copy(cent_v.at[j],
                             out_hbm.at[b // 8, h * _JROWS + j, b % 8],
                             sem_o))
    for c in out_rows:
        c.wait()
    ci.wait()


@functools.lru_cache(maxsize=1)
def _rvs_kernel():
    mesh = plsc.VectorSubcoreMesh(core_axis_name="c", subcore_axis_name="s")
    return pl.kernel(
        _rvs_body,
        out_type=(jax.ShapeDtypeStruct((_B // 8, _NPOINTS // 128, 8, 128),
                                       jnp.int32),
                  jax.ShapeDtypeStruct((_B, _NPOINTS // 128, 4, 128),
                                       jnp.int32)),
        mesh=mesh,
        scratch_types=[
            pltpu.VMEM((3 * _CHUNK,), jnp.int32),
            pltpu.VMEM((_V,), jnp.int32),
            pltpu.VMEM((_JROWS, 128), jnp.int32),
            pltpu.VMEM((_JROWS, 128), jnp.int32),
            pltpu.SemaphoreType.DMA,
            pltpu.SemaphoreType.DMA,
            pltpu.SemaphoreType.DMA,
            pltpu.SemaphoreType.DMA,
        ] + [pltpu.SemaphoreType.DMA] * _JROWS,
        name="rvs_gather",
        compiler_params=pltpu.CompilerParams(needs_layout_passes=False),
    )


def kernel(pos, voxel_points, voxel_counts):
    del pos
    const_tab, cidx, cidx4 = _rvs_constants()
    del cidx
    vp_t = jnp.transpose(voxel_points, (2, 0, 1))
    vp_flat = jnp.transpose(
        vp_t.reshape(_P, _B // 8, 8, _V // 128, 128),
        (0, 1, 3, 2, 4)).reshape(_P * _B * _V)
    cents4, oidx4 = _rvs_kernel()(voxel_counts, vp_flat,
                                  jnp.asarray(const_tab), jnp.asarray(cidx4))
    centroids = cents4.transpose(0, 2, 1, 3).reshape(_B, _NPOINTS)
    centroids_index = oidx4.transpose(0, 1, 3, 2).reshape(_B, _NPOINTS, 4)
    return centroids, centroids_index


# speedup vs baseline: 1.2583x; 1.2583x over previous
import functools

import jax
import jax.numpy as jnp
import numpy as np
from jax import lax
from jax.experimental import pallas as pl
from jax.experimental.pallas import tpu as pltpu
from jax.experimental.pallas import tpu_sc as plsc

_NPOINTS = 1024
_VS = 16
_V = _VS ** 3
_P = 8
_B = 16
_NW = 32
_CHUNK = _B * _NPOINTS // _NW
_JROWS = _CHUNK // 128


def _threefry2x32(kpair, count):
    odd = count.size % 2
    flat = count.astype(np.uint32).ravel()
    if odd:
        flat = np.concatenate([flat, np.zeros(1, np.uint32)])
    half = flat.size // 2
    x0, x1 = flat[:half].copy(), flat[half:].copy()
    k0, k1 = np.uint32(kpair[0]), np.uint32(kpair[1])
    ks = [k0, k1, k0 ^ k1 ^ np.uint32(0x1BD11BDA)]
    rotations = ((13, 15, 26, 6), (17, 29, 16, 24))
    with np.errstate(over="ignore"):
        x0 += ks[0]
        x1 += ks[1]
        for i in range(5):
            for r in rotations[i % 2]:
                x0 += x1
                x1 = (x1 << np.uint32(r)) | (x1 >> np.uint32(32 - r))
                x1 ^= x0
            x0 += ks[(i + 1) % 3]
            x1 += ks[(i + 2) % 3] + np.uint32(i + 1)
    out = np.concatenate([x0, x1])
    if odd:
        out = out[:-1]
    return out.reshape(count.shape)


def _fry_blocks(kpair, n):
    cnt = np.concatenate([np.zeros(n, np.uint32), np.arange(n, dtype=np.uint32)])
    out = _threefry2x32(kpair, cnt)
    return out[:n], out[n:]


def _random_bits(kpair, shape):
    size = int(np.prod(shape))
    hi, lo = _fry_blocks(kpair, size)
    return (hi ^ lo).reshape(shape)


def _split(kpair, n=2):
    hi, lo = _fry_blocks(kpair, n)
    return np.stack([hi, lo], axis=-1)


@functools.lru_cache(maxsize=1)
def _rvs_constants():
    root = (np.uint32(0), np.uint32(42))
    keys = _split(root, _B)
    sel = np.empty((_B, _NPOINTS), np.int32)
    r = np.empty((_B, _NPOINTS), np.int32)
    span = np.uint32(2 ** 31 - 1)
    with np.errstate(over="ignore"):
        m16 = np.uint32(np.uint32(65536) % span)
        mult = np.uint32(np.uint32(m16 * m16) % span)
    for b in range(_B):
        kv, kp = _split(keys[b])
        bits = _random_bits(kv, (_V,))
        scores = ((bits >> np.uint32(9)) | np.uint32(0x3F800000)).view(
            np.float32) - np.float32(1.0)
        order = np.argsort(scores, kind="stable")
        sel[b] = order[:_NPOINTS].astype(np.int32)
        khi, klo = _split(kp)
        u = _random_bits(khi, (_NPOINTS,))
        v = _random_bits(klo, (_NPOINTS,))
        with np.errstate(over="ignore"):
            off = np.uint32((u % span) * mult + v % span) % span
        r[b] = off.astype(np.int32)

    bb = np.arange(_B, dtype=np.int32)[:, None]
    phys = (((bb // 8) * (_V // 128) + sel // 128) * 1024
            + (bb % 8) * 128 + sel % 128)
    slotpack = np.zeros_like(r)
    for c in range(1, _P + 1):
        slotpack |= (r % c) << (3 * (c - 1))
    const_rows = np.stack([phys.reshape(_NW, _CHUNK),
                           sel.reshape(_NW, _CHUNK),
                           slotpack.reshape(_NW, _CHUNK)], axis=1)

    kx = sel // (_VS * _VS)
    ky = (sel // _VS) % _VS
    kz = sel % _VS
    bids = np.broadcast_to(np.arange(_B, dtype=np.int32)[:, None], (_B, _NPOINTS))
    cidx = np.stack([bids, kx, ky, kz], axis=-1).astype(np.int32)
    const_tab = const_rows.reshape(_NW, 3 * _CHUNK)
    cidx4 = cidx.reshape(_B, _NPOINTS // 128, 128, 4).transpose(0, 1, 3, 2)
    cidx4 = np.ascontiguousarray(cidx4)
    return const_tab, cidx, cidx4


def _rvs_body(vc_hbm, vp_hbm, const_hbm, out_hbm,
              cr_v, vcs_v, fidx_v, cent_v, sem_k, sem_v, sem_o, *sems):
    wid = lax.axis_index("s") * 2 + lax.axis_index("c")
    b = wid // 2
    h = wid - 2 * b

    ck = pltpu.async_copy(const_hbm.at[wid], cr_v, sem_k)
    cv = pltpu.async_copy(vc_hbm.at[b], vcs_v, sem_v)
    ck.wait()
    cv.wait()

    cent_copies = []
    for j in range(_JROWS):
        for i in range(8):
            pt = j * 128 + i * 16
            cnt = plsc.load_gather(vcs_v, [cr_v[pl.ds(_CHUNK + pt, 16)]])
            pack = cr_v[pl.ds(2 * _CHUNK + pt, 16)]
            slot = lax.shift_right_logical(pack, cnt * 3 - 3) & 7
            fidx_v[j, pl.ds(i * 16, 16)] = (slot * (_B * _V)
                                            + cr_v[pl.ds(pt, 16)])
        cent_copies.append(
            pltpu.async_copy(vp_hbm.at[fidx_v.at[j]], cent_v.at[j], sems[j]))

    out_rows = []
    for j in range(_JROWS):
        cent_copies[j].wait()
        out_rows.append(
            pltpu.async_copy(cent_v.at[j],
                             out_hbm.at[b // 8, h * _JROWS + j, b % 8],
                             sem_o))
    for c in out_rows:
        c.wait()


@functools.lru_cache(maxsize=1)
def _rvs_kernel():
    mesh = plsc.VectorSubcoreMesh(core_axis_name="c", subcore_axis_name="s")
    return pl.kernel(
        _rvs_body,
        out_type=jax.ShapeDtypeStruct((_B // 8, _NPOINTS // 128, 8, 128),
                                      jnp.int32),
        mesh=mesh,
        scratch_types=[
            pltpu.VMEM((3 * _CHUNK,), jnp.int32),
            pltpu.VMEM((_V,), jnp.int32),
            pltpu.VMEM((_JROWS, 128), jnp.int32),
            pltpu.VMEM((_JROWS, 128), jnp.int32),
            pltpu.SemaphoreType.DMA,
            pltpu.SemaphoreType.DMA,
            pltpu.SemaphoreType.DMA,
        ] + [pltpu.SemaphoreType.DMA] * _JROWS,
        name="rvs_gather",
        compiler_params=pltpu.CompilerParams(needs_layout_passes=False),
    )


def kernel(pos, voxel_points, voxel_counts):
    del pos
    const_tab, cidx, _cidx4 = _rvs_constants()
    vp_t = jnp.transpose(voxel_points, (2, 0, 1))
    vp_flat = jnp.transpose(
        vp_t.reshape(_P, _B // 8, 8, _V // 128, 128),
        (0, 1, 3, 2, 4)).reshape(_P * _B * _V)
    cents4 = _rvs_kernel()(voxel_counts, vp_flat, jnp.asarray(const_tab))
    centroids = cents4.transpose(0, 2, 1, 3).reshape(_B, _NPOINTS)
    centroids_index = jnp.asarray(cidx)
    return centroids, centroids_index


# speedup vs baseline: 1.3313x; 1.0580x over previous
import functools

import jax
import jax.numpy as jnp
import numpy as np
from jax import lax
from jax.experimental import pallas as pl
from jax.experimental.pallas import tpu as pltpu
from jax.experimental.pallas import tpu_sc as plsc

_NPOINTS = 1024
_VS = 16
_V = _VS ** 3
_P = 8
_B = 16
_NW = 32
_CHUNK = _B * _NPOINTS // _NW
_JROWS = _CHUNK // 128


def _threefry2x32(kpair, count):
    odd = count.size % 2
    flat = count.astype(np.uint32).ravel()
    if odd:
        flat = np.concatenate([flat, np.zeros(1, np.uint32)])
    half = flat.size // 2
    x0, x1 = flat[:half].copy(), flat[half:].copy()
    k0, k1 = np.uint32(kpair[0]), np.uint32(kpair[1])
    ks = [k0, k1, k0 ^ k1 ^ np.uint32(0x1BD11BDA)]
    rotations = ((13, 15, 26, 6), (17, 29, 16, 24))
    with np.errstate(over="ignore"):
        x0 += ks[0]
        x1 += ks[1]
        for i in range(5):
            for r in rotations[i % 2]:
                x0 += x1
                x1 = (x1 << np.uint32(r)) | (x1 >> np.uint32(32 - r))
                x1 ^= x0
            x0 += ks[(i + 1) % 3]
            x1 += ks[(i + 2) % 3] + np.uint32(i + 1)
    out = np.concatenate([x0, x1])
    if odd:
        out = out[:-1]
    return out.reshape(count.shape)


def _fry_blocks(kpair, n):
    cnt = np.concatenate([np.zeros(n, np.uint32), np.arange(n, dtype=np.uint32)])
    out = _threefry2x32(kpair, cnt)
    return out[:n], out[n:]


def _random_bits(kpair, shape):
    size = int(np.prod(shape))
    hi, lo = _fry_blocks(kpair, size)
    return (hi ^ lo).reshape(shape)


def _split(kpair, n=2):
    hi, lo = _fry_blocks(kpair, n)
    return np.stack([hi, lo], axis=-1)


@functools.lru_cache(maxsize=1)
def _rvs_constants():
    root = (np.uint32(0), np.uint32(42))
    keys = _split(root, _B)
    sel = np.empty((_B, _NPOINTS), np.int32)
    r = np.empty((_B, _NPOINTS), np.int32)
    span = np.uint32(2 ** 31 - 1)
    with np.errstate(over="ignore"):
        m16 = np.uint32(np.uint32(65536) % span)
        mult = np.uint32(np.uint32(m16 * m16) % span)
    for b in range(_B):
        kv, kp = _split(keys[b])
        bits = _random_bits(kv, (_V,))
        scores = ((bits >> np.uint32(9)) | np.uint32(0x3F800000)).view(
            np.float32) - np.float32(1.0)
        order = np.argsort(scores, kind="stable")
        sel[b] = order[:_NPOINTS].astype(np.int32)
        khi, klo = _split(kp)
        u = _random_bits(khi, (_NPOINTS,))
        v = _random_bits(klo, (_NPOINTS,))
        with np.errstate(over="ignore"):
            off = np.uint32((u % span) * mult + v % span) % span
        r[b] = off.astype(np.int32)

    bb = np.arange(_B, dtype=np.int32)[:, None]
    phys = (bb * (_V // 128) + sel // 128) * (_P * 128) + sel % 128
    slotpack = np.zeros_like(r)
    for c in range(1, _P + 1):
        slotpack |= (r % c) << (3 * (c - 1))
    const_rows = np.stack([phys.reshape(_NW, _CHUNK),
                           sel.reshape(_NW, _CHUNK),
                           slotpack.reshape(_NW, _CHUNK)], axis=1)

    kx = sel // (_VS * _VS)
    ky = (sel // _VS) % _VS
    kz = sel % _VS
    bids = np.broadcast_to(np.arange(_B, dtype=np.int32)[:, None], (_B, _NPOINTS))
    cidx = np.stack([bids, kx, ky, kz], axis=-1).astype(np.int32)
    const_tab = const_rows.reshape(_NW, 3 * _CHUNK)
    cidx4 = cidx.reshape(_B, _NPOINTS // 128, 128, 4).transpose(0, 1, 3, 2)
    cidx4 = np.ascontiguousarray(cidx4)
    return const_tab, cidx, cidx4


def _rvs_body(vc_hbm, vp_hbm, const_hbm, out_hbm,
              cr_v, vcs_v, fidx_v, cent_v, sem_k, sem_v, sem_o, *sems):
    wid = lax.axis_index("s") * 2 + lax.axis_index("c")
    b = wid // 2
    h = wid - 2 * b

    ck = pltpu.async_copy(const_hbm.at[wid], cr_v, sem_k)
    cv = pltpu.async_copy(vc_hbm.at[b], vcs_v, sem_v)
    ck.wait()
    cv.wait()

    cent_copies = []
    for j in range(_JROWS):
        for i in range(8):
            pt = j * 128 + i * 16
            cnt = plsc.load_gather(vcs_v, [cr_v[pl.ds(_CHUNK + pt, 16)]])
            pack = cr_v[pl.ds(2 * _CHUNK + pt, 16)]
            slot = lax.shift_right_logical(pack, cnt * 3 - 3) & 7
            fidx_v[j, pl.ds(i * 16, 16)] = (slot * 128
                                            + cr_v[pl.ds(pt, 16)])
        cent_copies.append(
            pltpu.async_copy(vp_hbm.at[fidx_v.at[j]], cent_v.at[j], sems[j]))

    out_rows = []
    for j in range(_JROWS):
        cent_copies[j].wait()
        out_rows.append(
            pltpu.async_copy(cent_v.at[j],
                             out_hbm.at[b // 8, h * _JROWS + j, b % 8],
                             sem_o))
    for c in out_rows:
        c.wait()


@functools.lru_cache(maxsize=1)
def _rvs_kernel():
    mesh = plsc.VectorSubcoreMesh(core_axis_name="c", subcore_axis_name="s")
    return pl.kernel(
        _rvs_body,
        out_type=jax.ShapeDtypeStruct((_B // 8, _NPOINTS // 128, 8, 128),
                                      jnp.int32),
        mesh=mesh,
        scratch_types=[
            pltpu.VMEM((3 * _CHUNK,), jnp.int32),
            pltpu.VMEM((_V,), jnp.int32),
            pltpu.VMEM((_JROWS, 128), jnp.int32),
            pltpu.VMEM((_JROWS, 128), jnp.int32),
            pltpu.SemaphoreType.DMA,
            pltpu.SemaphoreType.DMA,
            pltpu.SemaphoreType.DMA,
        ] + [pltpu.SemaphoreType.DMA] * _JROWS,
        name="rvs_gather",
        compiler_params=pltpu.CompilerParams(needs_layout_passes=False),
    )


def kernel(pos, voxel_points, voxel_counts):
    del pos
    const_tab, cidx, _cidx4 = _rvs_constants()
    vp_t = jnp.transpose(voxel_points, (2, 0, 1))
    vp_flat = jnp.transpose(
        vp_t.reshape(_P, _B, _V // 128, 128),
        (1, 2, 0, 3)).reshape(_P * _B * _V)
    cents4 = _rvs_kernel()(voxel_counts, vp_flat, jnp.asarray(const_tab))
    centroids = cents4.transpose(0, 2, 1, 3).reshape(_B, _NPOINTS)
    centroids_index = jnp.asarray(cidx)
    return centroids, centroids_index
